# baseline (speedup 1.0000x reference)
import functools

import jax
import jax.numpy as jnp
from jax import lax
from jax.experimental import pallas as pl
from jax.experimental.pallas import tpu as pltpu
from jax.experimental.pallas import tpu_sc as plsc

B = 16384
D = 64
V = 1000000
SUB = 8
G = V // SUB
NC = 2
NS = 16
NW = NC * NS
BPW = B // NW
L = 16
NGRP = BPW // L

_mesh = plsc.VectorSubcoreMesh(core_axis_name="c", subcore_axis_name="s")


@functools.partial(
    pl.kernel,
    mesh=_mesh,
    compiler_params=pltpu.CompilerParams(needs_layout_passes=False),
    out_type=jax.ShapeDtypeStruct((B,), jnp.float32),
    scratch_types=[
        pltpu.VMEM((BPW,), jnp.int32),
        pltpu.VMEM((BPW,), jnp.int32),
        pltpu.VMEM((2, L, D), jnp.float32),
        pltpu.VMEM((2, L, D), jnp.float32),
        pltpu.VMEM((BPW,), jnp.float32),
        pltpu.SemaphoreType.DMA,
        pltpu.SemaphoreType.DMA,
        pltpu.SemaphoreType.DMA,
        pltpu.SemaphoreType.DMA,
    ],
)
def _pointmf_sc(user_hbm, item_hbm, eu_hbm, ei_hbm, out_hbm,
                uidx, iidx, ubuf, ibuf, outv, su0, su1, si0, si1):
    wid = lax.axis_index("s") * NC + lax.axis_index("c")
    base = wid * BPW

    pltpu.sync_copy(user_hbm.at[pl.ds(base, BPW)], uidx)
    pltpu.sync_copy(item_hbm.at[pl.ds(base, BPW)], iidx)

    lanes = lax.iota(jnp.int32, L)
    sems = (su0, su1, si0, si1)

    def fire(g, slot):
        r0 = g * L
        uvec = uidx[pl.ds(r0, L)]
        ivec = iidx[pl.ds(r0, L)]
        gu = uvec >> 3
        gi = ivec >> 3
        hu = uvec & 7
        hi = ivec & 7
        for c in range(L):
            pltpu.async_copy(
                eu_hbm.at[gu[c], hu[c]], ubuf.at[slot, c], sems[slot])
            pltpu.async_copy(
                ei_hbm.at[gi[c], hi[c]], ibuf.at[slot, c], sems[2 + slot])

    def wait(slot):
        for c in range(L):
            pltpu.make_async_copy(
                eu_hbm.at[0, 0], ubuf.at[slot, 0], sems[slot]).wait()
            pltpu.make_async_copy(
                ei_hbm.at[0, 0], ibuf.at[slot, 0], sems[2 + slot]).wait()

    def compute(g, slot):
        r0 = g * L
        out_vec = jnp.zeros((L,), jnp.float32)
        for c in range(L):
            acc = None
            for k in range(D // L):
                u = ubuf[slot, c, pl.ds(k * L, L)]
                v = ibuf[slot, c, pl.ds(k * L, L)]
                p = u * v
                acc = p if acc is None else acc + p
            csum = plsc.cumsum(acc)
            bs = lax.broadcast(csum[L - 1], (L,))
            out_vec = jnp.where(lanes == c, bs, out_vec)
        outv[pl.ds(r0, L)] = out_vec

    fire(0, 0)

    def pair_body(p, carry):
        g0 = p * 2
        fire(g0 + 1, 1)
        wait(0)
        compute(g0, 0)

        @pl.when(g0 + 2 < NGRP)
        def _():
            fire(g0 + 2, 0)

        wait(1)
        compute(g0 + 1, 1)
        return carry

    lax.fori_loop(0, NGRP // 2, pair_body, 0)
    pltpu.sync_copy(outv, out_hbm.at[pl.ds(base, BPW)])


def _tc_transpose(tview):
    BLK = 8192

    def body(in_ref, out_ref):
        out_ref[...] = in_ref[...].T

    nblk = pl.cdiv(V, BLK)
    return pl.pallas_call(
        body,
        grid=(nblk,),
        in_specs=[pl.BlockSpec((D, BLK), lambda j: (0, j))],
        out_specs=pl.BlockSpec((BLK, D), lambda j: (j, 0)),
        out_shape=jax.ShapeDtypeStruct((V, D), jnp.float32),
    )(tview)


def kernel(user, item, embed_user, embed_item):
    eu3 = _tc_transpose(embed_user.T).reshape(G, SUB, D)
    ei3 = embed_item.reshape(G, SUB, D)
    return _pointmf_sc(user, item, eu3, ei3)


# speedup vs baseline: 1.0241x; 1.0241x over previous
import functools

import jax
import jax.numpy as jnp
from jax import lax
from jax.experimental import pallas as pl
from jax.experimental.pallas import tpu as pltpu
from jax.experimental.pallas import tpu_sc as plsc

B = 16384
D = 64
V = 1000000
SUB = 8
G = V // SUB
NC = 2
NS = 16
NW = NC * NS
BPW = B // NW
L = 16
NGRP = BPW // L

_mesh = plsc.VectorSubcoreMesh(core_axis_name="c", subcore_axis_name="s")


@functools.partial(
    pl.kernel,
    mesh=_mesh,
    compiler_params=pltpu.CompilerParams(needs_layout_passes=False),
    out_type=jax.ShapeDtypeStruct((B,), jnp.float32),
    scratch_types=[
        pltpu.VMEM((BPW,), jnp.int32),
        pltpu.VMEM((BPW,), jnp.int32),
        pltpu.VMEM((2, L, D), jnp.float32),
        pltpu.VMEM((2, L, D), jnp.float32),
        pltpu.VMEM((BPW,), jnp.float32),
        pltpu.SemaphoreType.DMA,
        pltpu.SemaphoreType.DMA,
        pltpu.SemaphoreType.DMA,
        pltpu.SemaphoreType.DMA,
    ],
)
def _pointmf_sc(user_hbm, item_hbm, eu_hbm, ei_hbm, out_hbm,
                uidx, iidx, ubuf, ibuf, outv, su0, su1, si0, si1):
    wid = lax.axis_index("s") * NC + lax.axis_index("c")
    base = wid * BPW

    pltpu.sync_copy(user_hbm.at[pl.ds(base, BPW)], uidx)
    pltpu.sync_copy(item_hbm.at[pl.ds(base, BPW)], iidx)

    lanes = lax.iota(jnp.int32, L)
    sems = (su0, su1, si0, si1)

    def fire(g, slot):
        r0 = g * L
        uvec = uidx[pl.ds(r0, L)]
        ivec = iidx[pl.ds(r0, L)]
        gu = uvec >> 3
        gi = ivec >> 3
        hu = uvec & 7
        hi = ivec & 7
        for c in range(L):
            pltpu.async_copy(
                eu_hbm.at[gu[c], hu[c]], ubuf.at[slot, c], sems[slot])
            pltpu.async_copy(
                ei_hbm.at[gi[c], hi[c]], ibuf.at[slot, c], sems[2 + slot])

    def wait(slot):
        for c in range(L):
            pltpu.make_async_copy(
                eu_hbm.at[0, 0], ubuf.at[slot, 0], sems[slot]).wait()
            pltpu.make_async_copy(
                ei_hbm.at[0, 0], ibuf.at[slot, 0], sems[2 + slot]).wait()

    def compute(g, slot):
        r0 = g * L
        out_vec = jnp.zeros((L,), jnp.float32)
        for c in range(L):
            acc = None
            for k in range(D // L):
                u = ubuf[slot, c, pl.ds(k * L, L)]
                v = ibuf[slot, c, pl.ds(k * L, L)]
                p = u * v
                acc = p if acc is None else acc + p
            csum = plsc.cumsum(acc)
            bs = lax.broadcast(csum[L - 1], (L,))
            out_vec = jnp.where(lanes == c, bs, out_vec)
        outv[pl.ds(r0, L)] = out_vec

    fire(0, 0)

    def pair_body(p, carry):
        g0 = p * 2
        fire(g0 + 1, 1)
        wait(0)
        compute(g0, 0)

        @pl.when(g0 + 2 < NGRP)
        def _():
            fire(g0 + 2, 0)

        wait(1)
        compute(g0 + 1, 1)
        return carry

    lax.fori_loop(0, NGRP // 2, pair_body, 0)
    pltpu.sync_copy(outv, out_hbm.at[pl.ds(base, BPW)])


def _tc_transpose(tview):
    BLK = 16384

    def body(in_ref, out_ref):
        out_ref[...] = in_ref[...].T

    nblk = pl.cdiv(V, BLK)
    return pl.pallas_call(
        body,
        grid=(nblk,),
        in_specs=[pl.BlockSpec((D, BLK), lambda j: (0, j))],
        out_specs=pl.BlockSpec((BLK, D), lambda j: (j, 0)),
        out_shape=jax.ShapeDtypeStruct((V, D), jnp.float32),
    )(tview)


def kernel(user, item, embed_user, embed_item):
    eu3 = _tc_transpose(embed_user.T).reshape(G, SUB, D)
    ei3 = embed_item.reshape(G, SUB, D)
    return _pointmf_sc(user, item, eu3, ei3)


# speedup vs baseline: 1.0371x; 1.0127x over previous
import functools

import jax
import jax.numpy as jnp
from jax import lax
from jax.experimental import pallas as pl
from jax.experimental.pallas import tpu as pltpu
from jax.experimental.pallas import tpu_sc as plsc

B = 16384
D = 64
V = 1000000
SUB = 8
G = V // SUB
NC = 2
NS = 16
NW = NC * NS
BPW = B // NW
L = 16
NGRP = BPW // L

_mesh = plsc.VectorSubcoreMesh(core_axis_name="c", subcore_axis_name="s")


@functools.partial(
    pl.kernel,
    mesh=_mesh,
    compiler_params=pltpu.CompilerParams(needs_layout_passes=False),
    out_type=jax.ShapeDtypeStruct((B,), jnp.float32),
    scratch_types=[
        pltpu.VMEM((BPW,), jnp.int32),
        pltpu.VMEM((BPW,), jnp.int32),
        pltpu.VMEM((2, L, D), jnp.float32),
        pltpu.VMEM((2, L, D), jnp.float32),
        pltpu.VMEM((BPW,), jnp.float32),
        pltpu.SemaphoreType.DMA,
        pltpu.SemaphoreType.DMA,
        pltpu.SemaphoreType.DMA,
        pltpu.SemaphoreType.DMA,
    ],
)
def _pointmf_sc(user_hbm, item_hbm, eu_hbm, ei_hbm, out_hbm,
                uidx, iidx, ubuf, ibuf, outv, su0, su1, si0, si1):
    wid = lax.axis_index("s") * NC + lax.axis_index("c")
    base = wid * BPW

    pltpu.sync_copy(user_hbm.at[pl.ds(base, BPW)], uidx)
    pltpu.sync_copy(item_hbm.at[pl.ds(base, BPW)], iidx)

    lanes = lax.iota(jnp.int32, L)
    sems = (su0, su1, si0, si1)

    def fire(g, slot):
        r0 = g * L
        uvec = uidx[pl.ds(r0, L)]
        ivec = iidx[pl.ds(r0, L)]
        gu = uvec >> 3
        gi = ivec >> 3
        hu = uvec & 7
        hi = ivec & 7
        for c in range(L):
            pltpu.async_copy(
                eu_hbm.at[gu[c], hu[c]], ubuf.at[slot, c], sems[slot])
            pltpu.async_copy(
                ei_hbm.at[gi[c], hi[c]], ibuf.at[slot, c], sems[2 + slot])

    def wait(slot):
        for c in range(L):
            pltpu.make_async_copy(
                eu_hbm.at[0, 0], ubuf.at[slot, 0], sems[slot]).wait()
            pltpu.make_async_copy(
                ei_hbm.at[0, 0], ibuf.at[slot, 0], sems[2 + slot]).wait()

    def compute(g, slot):
        r0 = g * L
        out_vec = jnp.zeros((L,), jnp.float32)
        for c in range(L):
            acc = None
            for k in range(D // L):
                u = ubuf[slot, c, pl.ds(k * L, L)]
                v = ibuf[slot, c, pl.ds(k * L, L)]
                p = u * v
                acc = p if acc is None else acc + p
            csum = plsc.cumsum(acc)
            bs = lax.broadcast(csum[L - 1], (L,))
            out_vec = jnp.where(lanes == c, bs, out_vec)
        outv[pl.ds(r0, L)] = out_vec

    fire(0, 0)

    def pair_body(p, carry):
        g0 = p * 2
        fire(g0 + 1, 1)
        wait(0)
        compute(g0, 0)

        @pl.when(g0 + 2 < NGRP)
        def _():
            fire(g0 + 2, 0)

        wait(1)
        compute(g0 + 1, 1)
        return carry

    lax.fori_loop(0, NGRP // 2, pair_body, 0)
    pltpu.sync_copy(outv, out_hbm.at[pl.ds(base, BPW)])


def _tc_transpose(tview):
    BLK = 32768

    def body(in_ref, out_ref):
        out_ref[...] = in_ref[...].T

    nblk = pl.cdiv(V, BLK)
    return pl.pallas_call(
        body,
        grid=(nblk,),
        in_specs=[pl.BlockSpec((D, BLK), lambda j: (0, j))],
        out_specs=pl.BlockSpec((BLK, D), lambda j: (j, 0)),
        out_shape=jax.ShapeDtypeStruct((V, D), jnp.float32),
    )(tview)


def kernel(user, item, embed_user, embed_item):
    eu3 = _tc_transpose(embed_user.T).reshape(G, SUB, D)
    ei3 = embed_item.reshape(G, SUB, D)
    return _pointmf_sc(user, item, eu3, ei3)
